# Initial kernel scaffold; baseline (speedup 1.0000x reference)
import jax
import jax.numpy as jnp
from jax.experimental import pallas as pl


def kernel(x, edge_index, reranker_scores, W_l, b_l, W_r, W_score, b_score, alpha):
    raise NotImplementedError("write your pallas kernel here")


# speedup vs baseline: 1.8818x; 1.8818x over previous
import functools

import jax
import jax.numpy as jnp
from jax import lax
from jax.experimental import pallas as pl
from jax.experimental.pallas import tpu as pltpu
from jax.experimental.pallas import tpu_sc as plsc

NC = 2
NS = 16
LANES = 16
K = 128
GB = 8


def _make_sc_segment_sum(n_pad, d, chunks):
    rows_per_tile = n_pad // NS
    zchunks = rows_per_tile // K
    ndeg = n_pad // 128

    mesh = plsc.VectorSubcoreMesh(core_axis_name="c", subcore_axis_name="s")

    @functools.partial(
        pl.kernel,
        mesh=mesh,
        out_type=(
            jax.ShapeDtypeStruct((NC, n_pad, d), jnp.float32),
            jax.ShapeDtypeStruct((NC, ndeg, 128), jnp.float32),
        ),
        scratch_types=(
            pltpu.VMEM((GB, K), jnp.int32),
            pltpu.VMEM((GB, K), jnp.int32),
            pltpu.VMEM((K,), jnp.int32),
            pltpu.VMEM((K,), jnp.int32),
            pltpu.VMEM((K,), jnp.int32),
            pltpu.VMEM((K,), jnp.int32),
            pltpu.VMEM((K, d), jnp.float32),
            pltpu.VMEM((K, 128), jnp.float32),
            pltpu.VMEM_SHARED((n_pad, d), jnp.float32),
            pltpu.VMEM_SHARED((ndeg, 128), jnp.float32),
            pltpu.SemaphoreType.DMA,
        ),
    )
    def sc_kernel(x_hbm, src_hbm, dst_hbm, oh_hbm, agg_out, deg_out,
                  src_g, dst_g, src_v, dst_v, lo_v, hi_v, rows_v, onerows_v,
                  sh_agg, sh_degw, sem):
        cid = lax.axis_index("c")
        sid = lax.axis_index("s")
        wid = sid * NC + cid

        zeros16 = jnp.zeros((LANES,), jnp.float32)

        def zrow(kk, _):
            for c in range(d // LANES):
                rows_v[kk, pl.ds(c * LANES, LANES)] = zeros16
            return _
        lax.fori_loop(0, K, zrow, None)

        row_base = sid * rows_per_tile
        for j in range(zchunks):
            pltpu.sync_copy(rows_v, sh_agg.at[pl.ds(row_base + j * K, K)])

        @pl.when(sid < ndeg // 8)
        def _():
            pltpu.sync_copy(rows_v.at[pl.ds(0, 8)], sh_degw.at[pl.ds(sid * 8, 8)])

        plsc.subcore_barrier()

        def body(go, _):
            goff = pl.multiple_of(go * GB, GB)
            pltpu.sync_copy(src_hbm.at[wid, pl.ds(goff, GB)], src_g)
            pltpu.sync_copy(dst_hbm.at[wid, pl.ds(goff, GB)], dst_g)
            for j in range(GB):
                for c in range(K // LANES):
                    sl = pl.ds(c * LANES, LANES)
                    src_v[sl] = src_g[j, sl]
                    dv = dst_g[j, sl]
                    dst_v[sl] = dv
                    lo_v[sl] = lax.rem(dv, 128)
                    hi_v[sl] = lax.div(dv, 128)
                pltpu.async_copy(x_hbm.at[src_v], rows_v, sem).wait()
                pltpu.sync_copy(rows_v, sh_agg.at[dst_v], add=True)
                pltpu.async_copy(oh_hbm.at[lo_v], onerows_v, sem).wait()
                pltpu.sync_copy(onerows_v, sh_degw.at[hi_v], add=True)
            return _
        lax.fori_loop(0, chunks // GB, body, None)

        plsc.subcore_barrier()

        for j in range(zchunks):
            rb = row_base + j * K
            pltpu.sync_copy(sh_agg.at[pl.ds(rb, K)], rows_v)
            pltpu.sync_copy(rows_v, agg_out.at[cid, pl.ds(rb, K)])

        @pl.when(sid < ndeg // 8)
        def _():
            pltpu.sync_copy(sh_degw.at[pl.ds(sid * 8, 8)], onerows_v.at[pl.ds(0, 8)])
            pltpu.sync_copy(onerows_v.at[pl.ds(0, 8)], deg_out.at[cid, pl.ds(sid * 8, 8)])

    return sc_kernel


def _tc_body(p_ref, degp_ref, x_ref, wl_ref, wr_ref, ws_ref, bl_ref, bs_ref,
             rr_ref, alpha_ref, out_ref):
    n_pad = p_ref.shape[1]
    ndeg = degp_ref.shape[1]
    p = p_ref[0] + p_ref[1]
    dgrid = degp_ref[0] + degp_ref[1]
    ri = lax.broadcasted_iota(jnp.int32, (n_pad, ndeg), 0)
    ci = lax.broadcasted_iota(jnp.int32, (n_pad, ndeg), 1)
    sel = jnp.where(ci == ri // 128, 1.0, 0.0)
    rep = lax.dot_general(sel, dgrid, (((1,), (0,)), ((), ())),
                          preferred_element_type=jnp.float32)
    li = lax.broadcasted_iota(jnp.int32, (n_pad, 128), 1)
    ro = lax.broadcasted_iota(jnp.int32, (n_pad, 128), 0)
    deg = jnp.sum(jnp.where(li == lax.rem(ro, 128), rep, 0.0),
                  axis=1, keepdims=True)
    agg = p / jnp.clip(deg, 1.0, None)
    dn = (((1,), (1,)), ((), ()))
    h = lax.dot_general(agg, wl_ref[...], dn, preferred_element_type=jnp.float32)
    h = h + lax.dot_general(x_ref[...], wr_ref[...], dn,
                            preferred_element_type=jnp.float32)
    h = jnp.maximum(h + bl_ref[...], 0.0)
    s = jnp.sum(h * ws_ref[...], axis=1, keepdims=True)
    s = s + bs_ref[0, 0]
    a = 1.0 / (1.0 + jnp.exp(-alpha_ref[0, 0]))
    out_ref[...] = a * rr_ref[...] + (1.0 - a) * s


def kernel(x, edge_index, reranker_scores, W_l, b_l, W_r, W_score, b_score, alpha):
    n, d = x.shape
    e = edge_index.shape[1]
    h_dim = W_l.shape[0]

    n_pad = ((n + 511) // 512) * 512
    eblk = NC * NS * K * GB
    e_pad = ((e + eblk - 1) // eblk) * eblk
    chunks = e_pad // (NC * NS * K)

    src = edge_index[0]
    dst = edge_index[1]
    pad = e_pad - e
    if pad:
        src = jnp.concatenate([src, jnp.zeros((pad,), jnp.int32)])
        dst = jnp.concatenate([dst, jnp.full((pad,), n, jnp.int32)])
    src3 = src.reshape(NC * NS, chunks, K)
    dst3 = dst.reshape(NC * NS, chunks, K)

    xp = jnp.zeros((n_pad, d), jnp.float32).at[:n, :].set(x)
    rrp = jnp.zeros((n_pad, 1), jnp.float32).at[:n, 0].set(reranker_scores)
    oh = jnp.eye(128, dtype=jnp.float32)

    sc = _make_sc_segment_sum(n_pad, d, chunks)
    partials, degs = sc(xp, src3, dst3, oh)

    out_pad = pl.pallas_call(
        _tc_body,
        out_shape=jax.ShapeDtypeStruct((n_pad, 1), jnp.float32),
    )(partials, degs, xp, W_l, W_r, W_score,
      b_l.reshape(1, h_dim), b_score.reshape(1, 1), rrp,
      jnp.asarray(alpha, jnp.float32).reshape(1, 1))

    return out_pad[:n, 0]


# speedup vs baseline: 3.0168x; 1.6031x over previous
import functools

import jax
import jax.numpy as jnp
from jax import lax
from jax.experimental import pallas as pl
from jax.experimental.pallas import tpu as pltpu
from jax.experimental.pallas import tpu_sc as plsc

NC = 2
NS = 16
LANES = 16
K = 128
GB = 8


def _make_sc_segment_sum(n_pad, d, chunks):
    rows_per_tile = n_pad // NS
    zchunks = rows_per_tile // K
    ndeg = n_pad // 128

    mesh = plsc.VectorSubcoreMesh(core_axis_name="c", subcore_axis_name="s")

    @functools.partial(
        pl.kernel,
        mesh=mesh,
        out_type=(
            jax.ShapeDtypeStruct((NC, n_pad, d), jnp.float32),
            jax.ShapeDtypeStruct((NC, ndeg, 128), jnp.float32),
        ),
        scratch_types=(
            pltpu.VMEM((GB, K), jnp.int32),
            pltpu.VMEM((GB, K), jnp.int32),
            pltpu.VMEM((K,), jnp.int32),
            pltpu.VMEM((K,), jnp.int32),
            pltpu.VMEM((K,), jnp.int32),
            pltpu.VMEM((K,), jnp.int32),
            pltpu.VMEM((K, d), jnp.float32),
            pltpu.VMEM((K, 128), jnp.float32),
            pltpu.VMEM_SHARED((n_pad, d), jnp.float32),
            pltpu.VMEM_SHARED((ndeg, 128), jnp.float32),
            pltpu.SemaphoreType.DMA,
            pltpu.SemaphoreType.DMA,
            pltpu.SemaphoreType.DMA,
            pltpu.SemaphoreType.DMA,
        ),
    )
    def sc_kernel(x_hbm, src_hbm, dst_hbm, oh_hbm, agg_out, deg_out,
                  src_g, dst_g, src_v, dst_v, lo_v, hi_v, rows_v, onerows_v,
                  sh_agg, sh_degw, sem, sem_goh, sem_sx, sem_soh):
        cid = lax.axis_index("c")
        sid = lax.axis_index("s")
        wid = sid * NC + cid

        zeros16 = jnp.zeros((LANES,), jnp.float32)

        def zrow(kk, _):
            for c in range(d // LANES):
                rows_v[kk, pl.ds(c * LANES, LANES)] = zeros16
            return _
        lax.fori_loop(0, K, zrow, None)

        row_base = sid * rows_per_tile
        for j in range(zchunks):
            pltpu.sync_copy(rows_v, sh_agg.at[pl.ds(row_base + j * K, K)])

        @pl.when(sid < ndeg // 8)
        def _():
            pltpu.sync_copy(rows_v.at[pl.ds(0, 8)], sh_degw.at[pl.ds(sid * 8, 8)])

        plsc.subcore_barrier()

        def body(go, _):
            goff = pl.multiple_of(go * GB, GB)
            pltpu.sync_copy(src_hbm.at[wid, pl.ds(goff, GB)], src_g)
            pltpu.sync_copy(dst_hbm.at[wid, pl.ds(goff, GB)], dst_g)
            for j in range(GB):
                for c in range(K // LANES):
                    sl = pl.ds(c * LANES, LANES)
                    src_v[sl] = src_g[j, sl]
                    dv = dst_g[j, sl]
                    dst_v[sl] = dv
                    lo_v[sl] = lax.rem(dv, 128)
                    hi_v[sl] = lax.div(dv, 128)
                gx = pltpu.async_copy(x_hbm.at[src_v], rows_v, sem)
                goh = pltpu.async_copy(oh_hbm.at[lo_v], onerows_v, sem_goh)
                gx.wait()
                sx = pltpu.async_copy(rows_v, sh_agg.at[dst_v], sem_sx, add=True)
                goh.wait()
                soh = pltpu.async_copy(onerows_v, sh_degw.at[hi_v], sem_soh,
                                       add=True)
                sx.wait()
                soh.wait()
            return _
        lax.fori_loop(0, chunks // GB, body, None)

        plsc.subcore_barrier()

        for j in range(zchunks):
            rb = row_base + j * K
            pltpu.sync_copy(sh_agg.at[pl.ds(rb, K)], rows_v)
            pltpu.sync_copy(rows_v, agg_out.at[cid, pl.ds(rb, K)])

        @pl.when(sid < ndeg // 8)
        def _():
            pltpu.sync_copy(sh_degw.at[pl.ds(sid * 8, 8)], onerows_v.at[pl.ds(0, 8)])
            pltpu.sync_copy(onerows_v.at[pl.ds(0, 8)], deg_out.at[cid, pl.ds(sid * 8, 8)])

    return sc_kernel


def _tc_body(p_ref, degp_ref, x_ref, wl_ref, wr_ref, ws_ref, bl_ref, bs_ref,
             rr_ref, alpha_ref, out_ref):
    n_pad = p_ref.shape[1]
    ndeg = degp_ref.shape[1]
    p = p_ref[0] + p_ref[1]
    dgrid = degp_ref[0] + degp_ref[1]
    ri = lax.broadcasted_iota(jnp.int32, (n_pad, ndeg), 0)
    ci = lax.broadcasted_iota(jnp.int32, (n_pad, ndeg), 1)
    sel = jnp.where(ci == ri // 128, 1.0, 0.0)
    rep = lax.dot_general(sel, dgrid, (((1,), (0,)), ((), ())),
                          preferred_element_type=jnp.float32)
    li = lax.broadcasted_iota(jnp.int32, (n_pad, 128), 1)
    ro = lax.broadcasted_iota(jnp.int32, (n_pad, 128), 0)
    deg = jnp.sum(jnp.where(li == lax.rem(ro, 128), rep, 0.0),
                  axis=1, keepdims=True)
    agg = p / jnp.clip(deg, 1.0, None)
    dn = (((1,), (1,)), ((), ()))
    h = lax.dot_general(agg, wl_ref[...], dn, preferred_element_type=jnp.float32)
    h = h + lax.dot_general(x_ref[...], wr_ref[...], dn,
                            preferred_element_type=jnp.float32)
    h = jnp.maximum(h + bl_ref[...], 0.0)
    s = jnp.sum(h * ws_ref[...], axis=1, keepdims=True)
    s = s + bs_ref[0, 0]
    a = 1.0 / (1.0 + jnp.exp(-alpha_ref[0, 0]))
    out_ref[...] = a * rr_ref[...] + (1.0 - a) * s


def kernel(x, edge_index, reranker_scores, W_l, b_l, W_r, W_score, b_score, alpha):
    n, d = x.shape
    e = edge_index.shape[1]
    h_dim = W_l.shape[0]

    n_pad = ((n + 511) // 512) * 512
    eblk = NC * NS * K * GB
    e_pad = ((e + eblk - 1) // eblk) * eblk
    chunks = e_pad // (NC * NS * K)

    src = edge_index[0]
    dst = edge_index[1]
    pad = e_pad - e
    if pad:
        src = jnp.concatenate([src, jnp.zeros((pad,), jnp.int32)])
        dst = jnp.concatenate([dst, jnp.full((pad,), n, jnp.int32)])
    src3 = src.reshape(NC * NS, chunks, K)
    dst3 = dst.reshape(NC * NS, chunks, K)

    xp = jnp.zeros((n_pad, d), jnp.float32).at[:n, :].set(x)
    rrp = jnp.zeros((n_pad, 1), jnp.float32).at[:n, 0].set(reranker_scores)
    oh = jnp.eye(128, dtype=jnp.float32)

    sc = _make_sc_segment_sum(n_pad, d, chunks)
    partials, degs = sc(xp, src3, dst3, oh)

    out_pad = pl.pallas_call(
        _tc_body,
        out_shape=jax.ShapeDtypeStruct((n_pad, 1), jnp.float32),
    )(partials, degs, xp, W_l, W_r, W_score,
      b_l.reshape(1, h_dim), b_score.reshape(1, 1), rrp,
      jnp.asarray(alpha, jnp.float32).reshape(1, 1))

    return out_pad[:n, 0]


# speedup vs baseline: 3.0799x; 1.0209x over previous
import functools

import jax
import jax.numpy as jnp
from jax import lax
from jax.experimental import pallas as pl
from jax.experimental.pallas import tpu as pltpu
from jax.experimental.pallas import tpu_sc as plsc

NC = 2
NS = 16
LANES = 16
K = 128
GB = 8


def _make_sc_segment_sum(n_pad, d, chunks):
    rows_per_tile = n_pad // NS
    zchunks = rows_per_tile // K
    ndeg = n_pad // 128

    mesh = plsc.VectorSubcoreMesh(core_axis_name="c", subcore_axis_name="s")

    @functools.partial(
        pl.kernel,
        mesh=mesh,
        out_type=(
            jax.ShapeDtypeStruct((NC, n_pad, d), jnp.float32),
            jax.ShapeDtypeStruct((NC, ndeg, 128), jnp.float32),
        ),
        scratch_types=(
            pltpu.VMEM((GB, K), jnp.int32),
            pltpu.VMEM((GB, K), jnp.int32),
            pltpu.VMEM((K,), jnp.int32),
            pltpu.VMEM((K,), jnp.int32),
            pltpu.VMEM((K,), jnp.int32),
            pltpu.VMEM((K,), jnp.int32),
            pltpu.VMEM((K, d), jnp.float32),
            pltpu.VMEM((K, 128), jnp.float32),
            pltpu.VMEM_SHARED((n_pad, d), jnp.float32),
            pltpu.VMEM_SHARED((ndeg, 128), jnp.float32),
            pltpu.VMEM_SHARED((128, 128), jnp.float32),
            pltpu.SemaphoreType.DMA,
            pltpu.SemaphoreType.DMA,
            pltpu.SemaphoreType.DMA,
            pltpu.SemaphoreType.DMA,
        ),
    )
    def sc_kernel(x_hbm, src_hbm, dst_hbm, oh_hbm, agg_out, deg_out,
                  src_g, dst_g, src_v, dst_v, lo_v, hi_v, rows_v, onerows_v,
                  sh_agg, sh_degw, sh_eye, sem, sem_goh, sem_sx, sem_soh):
        cid = lax.axis_index("c")
        sid = lax.axis_index("s")
        wid = sid * NC + cid

        zeros16 = jnp.zeros((LANES,), jnp.float32)

        def zrow(kk, _):
            for c in range(d // LANES):
                rows_v[kk, pl.ds(c * LANES, LANES)] = zeros16
            return _
        lax.fori_loop(0, K, zrow, None)

        row_base = sid * rows_per_tile
        for j in range(zchunks):
            pltpu.sync_copy(rows_v, sh_agg.at[pl.ds(row_base + j * K, K)])

        @pl.when(sid < ndeg // 8)
        def _():
            pltpu.sync_copy(rows_v.at[pl.ds(0, 8)], sh_degw.at[pl.ds(sid * 8, 8)])

        @pl.when(sid == NS - 1)
        def _():
            pltpu.sync_copy(oh_hbm, onerows_v)
            pltpu.sync_copy(onerows_v, sh_eye)

        plsc.subcore_barrier()

        def body(go, _):
            goff = pl.multiple_of(go * GB, GB)
            pltpu.sync_copy(src_hbm.at[wid, pl.ds(goff, GB)], src_g)
            pltpu.sync_copy(dst_hbm.at[wid, pl.ds(goff, GB)], dst_g)
            for j in range(GB):
                for c in range(K // LANES):
                    sl = pl.ds(c * LANES, LANES)
                    src_v[sl] = src_g[j, sl]
                    dv = dst_g[j, sl]
                    dst_v[sl] = dv
                    lo_v[sl] = lax.rem(dv, 128)
                    hi_v[sl] = lax.div(dv, 128)
                gx = pltpu.async_copy(x_hbm.at[src_v], rows_v, sem)
                goh = pltpu.async_copy(sh_eye.at[lo_v], onerows_v, sem_goh)
                gx.wait()
                sx = pltpu.async_copy(rows_v, sh_agg.at[dst_v], sem_sx, add=True)
                goh.wait()
                soh = pltpu.async_copy(onerows_v, sh_degw.at[hi_v], sem_soh,
                                       add=True)
                sx.wait()
                soh.wait()
            return _
        lax.fori_loop(0, chunks // GB, body, None)

        plsc.subcore_barrier()

        for j in range(zchunks):
            rb = row_base + j * K
            pltpu.sync_copy(sh_agg.at[pl.ds(rb, K)], rows_v)
            pltpu.sync_copy(rows_v, agg_out.at[cid, pl.ds(rb, K)])

        @pl.when(sid < ndeg // 8)
        def _():
            pltpu.sync_copy(sh_degw.at[pl.ds(sid * 8, 8)], onerows_v.at[pl.ds(0, 8)])
            pltpu.sync_copy(onerows_v.at[pl.ds(0, 8)], deg_out.at[cid, pl.ds(sid * 8, 8)])

    return sc_kernel


def _tc_body(p_ref, degp_ref, x_ref, wl_ref, wr_ref, ws_ref, bl_ref, bs_ref,
             rr_ref, alpha_ref, out_ref):
    n_pad = p_ref.shape[1]
    ndeg = degp_ref.shape[1]
    p = p_ref[0] + p_ref[1]
    dgrid = degp_ref[0] + degp_ref[1]
    ri = lax.broadcasted_iota(jnp.int32, (n_pad, ndeg), 0)
    ci = lax.broadcasted_iota(jnp.int32, (n_pad, ndeg), 1)
    sel = jnp.where(ci == ri // 128, 1.0, 0.0)
    rep = lax.dot_general(sel, dgrid, (((1,), (0,)), ((), ())),
                          preferred_element_type=jnp.float32)
    li = lax.broadcasted_iota(jnp.int32, (n_pad, 128), 1)
    ro = lax.broadcasted_iota(jnp.int32, (n_pad, 128), 0)
    deg = jnp.sum(jnp.where(li == lax.rem(ro, 128), rep, 0.0),
                  axis=1, keepdims=True)
    agg = p / jnp.clip(deg, 1.0, None)
    dn = (((1,), (1,)), ((), ()))
    h = lax.dot_general(agg, wl_ref[...], dn, preferred_element_type=jnp.float32)
    h = h + lax.dot_general(x_ref[...], wr_ref[...], dn,
                            preferred_element_type=jnp.float32)
    h = jnp.maximum(h + bl_ref[...], 0.0)
    s = jnp.sum(h * ws_ref[...], axis=1, keepdims=True)
    s = s + bs_ref[0, 0]
    a = 1.0 / (1.0 + jnp.exp(-alpha_ref[0, 0]))
    out_ref[...] = a * rr_ref[...] + (1.0 - a) * s


def kernel(x, edge_index, reranker_scores, W_l, b_l, W_r, W_score, b_score, alpha):
    n, d = x.shape
    e = edge_index.shape[1]
    h_dim = W_l.shape[0]

    n_pad = ((n + 511) // 512) * 512
    eblk = NC * NS * K * GB
    e_pad = ((e + eblk - 1) // eblk) * eblk
    chunks = e_pad // (NC * NS * K)

    src = edge_index[0]
    dst = edge_index[1]
    pad = e_pad - e
    if pad:
        src = jnp.concatenate([src, jnp.zeros((pad,), jnp.int32)])
        dst = jnp.concatenate([dst, jnp.full((pad,), n, jnp.int32)])
    src3 = src.reshape(NC * NS, chunks, K)
    dst3 = dst.reshape(NC * NS, chunks, K)

    xp = jnp.zeros((n_pad, d), jnp.float32).at[:n, :].set(x)
    rrp = jnp.zeros((n_pad, 1), jnp.float32).at[:n, 0].set(reranker_scores)
    oh = jnp.eye(128, dtype=jnp.float32)

    sc = _make_sc_segment_sum(n_pad, d, chunks)
    partials, degs = sc(xp, src3, dst3, oh)

    out_pad = pl.pallas_call(
        _tc_body,
        out_shape=jax.ShapeDtypeStruct((n_pad, 1), jnp.float32),
    )(partials, degs, xp, W_l, W_r, W_score,
      b_l.reshape(1, h_dim), b_score.reshape(1, 1), rrp,
      jnp.asarray(alpha, jnp.float32).reshape(1, 1))

    return out_pad[:n, 0]


# speedup vs baseline: 3.6152x; 1.1738x over previous
import functools

import jax
import jax.numpy as jnp
from jax import lax
from jax.experimental import pallas as pl
from jax.experimental.pallas import tpu as pltpu
from jax.experimental.pallas import tpu_sc as plsc

NC = 2
NS = 16
LANES = 16
K = 128
GB = 8


def _make_sc_segment_sum(n_pad, d, chunks):
    rows_per_tile = n_pad // NS
    zchunks = rows_per_tile // K
    ndeg = n_pad // 128

    mesh = plsc.VectorSubcoreMesh(core_axis_name="c", subcore_axis_name="s")

    @functools.partial(
        pl.kernel,
        mesh=mesh,
        out_type=(
            jax.ShapeDtypeStruct((NC, n_pad, d), jnp.float32),
            jax.ShapeDtypeStruct((NC, ndeg, 128), jnp.float32),
        ),
        scratch_types=(
            pltpu.VMEM((GB, K), jnp.int32),
            pltpu.VMEM((GB, K), jnp.int32),
            pltpu.VMEM((K // 2,), jnp.int32),
            pltpu.VMEM((K // 2,), jnp.int32),
            pltpu.VMEM((K // 2,), jnp.int32),
            pltpu.VMEM((K // 2,), jnp.int32),
            pltpu.VMEM((K // 2,), jnp.int32),
            pltpu.VMEM((K // 2,), jnp.int32),
            pltpu.VMEM((K // 2,), jnp.int32),
            pltpu.VMEM((K // 2,), jnp.int32),
            pltpu.VMEM((K // 2, d), jnp.float32),
            pltpu.VMEM((K // 2, d), jnp.float32),
            pltpu.VMEM((K // 2, 128), jnp.float32),
            pltpu.VMEM((K // 2, 128), jnp.float32),
            pltpu.VMEM_SHARED((n_pad, d), jnp.float32),
            pltpu.VMEM_SHARED((ndeg, 128), jnp.float32),
            pltpu.VMEM_SHARED((128, 128), jnp.float32),
            pltpu.SemaphoreType.DMA,
            pltpu.SemaphoreType.DMA,
            pltpu.SemaphoreType.DMA,
            pltpu.SemaphoreType.DMA,
        ),
    )
    def sc_kernel(x_hbm, src_hbm, dst_hbm, oh_hbm, agg_out, deg_out,
                  src_g, dst_g, src_v0, src_v1, dst_v0, dst_v1,
                  lo_v0, lo_v1, hi_v0, hi_v1, rows_v0, rows_v1,
                  onerows_v0, onerows_v1,
                  sh_agg, sh_degw, sh_eye, sem, sem_goh, sem_sx, sem_soh):
        src_vs = (src_v0, src_v1)
        dst_vs = (dst_v0, dst_v1)
        lo_vs = (lo_v0, lo_v1)
        hi_vs = (hi_v0, hi_v1)
        rows_vs = (rows_v0, rows_v1)
        onerows_vs = (onerows_v0, onerows_v1)
        cid = lax.axis_index("c")
        sid = lax.axis_index("s")
        wid = sid * NC + cid

        zeros16 = jnp.zeros((LANES,), jnp.float32)

        def zrow(kk, _):
            for c in range(d // LANES):
                rows_v0[kk, pl.ds(c * LANES, LANES)] = zeros16
            return _
        lax.fori_loop(0, K // 2, zrow, None)

        row_base = sid * rows_per_tile
        for j in range(2 * zchunks):
            pltpu.sync_copy(rows_v0,
                            sh_agg.at[pl.ds(row_base + j * (K // 2), K // 2)])

        @pl.when(sid < ndeg // 8)
        def _():
            pltpu.sync_copy(rows_v0.at[pl.ds(0, 8)], sh_degw.at[pl.ds(sid * 8, 8)])

        @pl.when(sid == NS - 1)
        def _():
            pltpu.sync_copy(oh_hbm.at[pl.ds(0, 64)], onerows_v0)
            pltpu.sync_copy(onerows_v0, sh_eye.at[pl.ds(0, 64)])
            pltpu.sync_copy(oh_hbm.at[pl.ds(64, 64)], onerows_v1)
            pltpu.sync_copy(onerows_v1, sh_eye.at[pl.ds(64, 64)])

        plsc.subcore_barrier()

        H = K // 2
        NH = 2 * GB

        def stage(hh, b):
            j, half = hh // 2, hh % 2
            for c in range(H // LANES):
                sl = pl.ds(c * LANES, LANES)
                gsl = pl.ds(half * H + c * LANES, LANES)
                src_vs[b][sl] = src_g[j, gsl]
                dv = dst_g[j, gsl]
                dst_vs[b][sl] = dv
                lo_vs[b][sl] = lax.rem(dv, 128)
                hi_vs[b][sl] = lax.div(dv, 128)

        def fire_gathers(b):
            gx = pltpu.async_copy(x_hbm.at[src_vs[b]], rows_vs[b], sem)
            goh = pltpu.async_copy(sh_eye.at[lo_vs[b]], onerows_vs[b], sem_goh)
            return gx, goh

        def fire_scatters(b):
            sx = pltpu.async_copy(rows_vs[b], sh_agg.at[dst_vs[b]],
                                  sem_sx, add=True)
            soh = pltpu.async_copy(onerows_vs[b], sh_degw.at[hi_vs[b]],
                                   sem_soh, add=True)
            return sx, soh

        def body(go, _):
            goff = pl.multiple_of(go * GB, GB)
            pltpu.sync_copy(src_hbm.at[wid, pl.ds(goff, GB)], src_g)
            pltpu.sync_copy(dst_hbm.at[wid, pl.ds(goff, GB)], dst_g)
            stage(0, 0)
            g = fire_gathers(0)
            s_prev = None
            for hh in range(NH):
                cur = hh % 2
                nxt = 1 - cur
                g[0].wait()
                g[1].wait()
                if s_prev is not None:
                    s_prev[0].wait()
                    s_prev[1].wait()
                if hh + 1 < NH:
                    stage(hh + 1, nxt)
                    g = fire_gathers(nxt)
                s_prev = fire_scatters(cur)
            s_prev[0].wait()
            s_prev[1].wait()
            return _
        lax.fori_loop(0, chunks // GB, body, None)

        plsc.subcore_barrier()

        for j in range(2 * zchunks):
            rb = row_base + j * (K // 2)
            pltpu.sync_copy(sh_agg.at[pl.ds(rb, K // 2)], rows_v0)
            pltpu.sync_copy(rows_v0, agg_out.at[cid, pl.ds(rb, K // 2)])

        @pl.when(sid < ndeg // 8)
        def _():
            pltpu.sync_copy(sh_degw.at[pl.ds(sid * 8, 8)], onerows_v0.at[pl.ds(0, 8)])
            pltpu.sync_copy(onerows_v0.at[pl.ds(0, 8)], deg_out.at[cid, pl.ds(sid * 8, 8)])

    return sc_kernel


def _tc_body(p_ref, degp_ref, x_ref, wl_ref, wr_ref, ws_ref, bl_ref, bs_ref,
             rr_ref, alpha_ref, out_ref):
    n_pad = p_ref.shape[1]
    ndeg = degp_ref.shape[1]
    p = p_ref[0] + p_ref[1]
    dgrid = degp_ref[0] + degp_ref[1]
    ri = lax.broadcasted_iota(jnp.int32, (n_pad, ndeg), 0)
    ci = lax.broadcasted_iota(jnp.int32, (n_pad, ndeg), 1)
    sel = jnp.where(ci == ri // 128, 1.0, 0.0)
    rep = lax.dot_general(sel, dgrid, (((1,), (0,)), ((), ())),
                          preferred_element_type=jnp.float32)
    li = lax.broadcasted_iota(jnp.int32, (n_pad, 128), 1)
    ro = lax.broadcasted_iota(jnp.int32, (n_pad, 128), 0)
    deg = jnp.sum(jnp.where(li == lax.rem(ro, 128), rep, 0.0),
                  axis=1, keepdims=True)
    agg = p / jnp.clip(deg, 1.0, None)
    dn = (((1,), (1,)), ((), ()))
    h = lax.dot_general(agg, wl_ref[...], dn, preferred_element_type=jnp.float32)
    h = h + lax.dot_general(x_ref[...], wr_ref[...], dn,
                            preferred_element_type=jnp.float32)
    h = jnp.maximum(h + bl_ref[...], 0.0)
    s = jnp.sum(h * ws_ref[...], axis=1, keepdims=True)
    s = s + bs_ref[0, 0]
    a = 1.0 / (1.0 + jnp.exp(-alpha_ref[0, 0]))
    out_ref[...] = a * rr_ref[...] + (1.0 - a) * s


def kernel(x, edge_index, reranker_scores, W_l, b_l, W_r, W_score, b_score, alpha):
    n, d = x.shape
    e = edge_index.shape[1]
    h_dim = W_l.shape[0]

    n_pad = ((n + 511) // 512) * 512
    eblk = NC * NS * K * GB
    e_pad = ((e + eblk - 1) // eblk) * eblk
    chunks = e_pad // (NC * NS * K)

    src = edge_index[0]
    dst = edge_index[1]
    pad = e_pad - e
    if pad:
        src = jnp.concatenate([src, jnp.zeros((pad,), jnp.int32)])
        dst = jnp.concatenate([dst, jnp.full((pad,), n, jnp.int32)])
    src3 = src.reshape(NC * NS, chunks, K)
    dst3 = dst.reshape(NC * NS, chunks, K)

    xp = jnp.zeros((n_pad, d), jnp.float32).at[:n, :].set(x)
    rrp = jnp.zeros((n_pad, 1), jnp.float32).at[:n, 0].set(reranker_scores)
    oh = jnp.eye(128, dtype=jnp.float32)

    sc = _make_sc_segment_sum(n_pad, d, chunks)
    partials, degs = sc(xp, src3, dst3, oh)

    out_pad = pl.pallas_call(
        _tc_body,
        out_shape=jax.ShapeDtypeStruct((n_pad, 1), jnp.float32),
    )(partials, degs, xp, W_l, W_r, W_score,
      b_l.reshape(1, h_dim), b_score.reshape(1, 1), rrp,
      jnp.asarray(alpha, jnp.float32).reshape(1, 1))

    return out_pad[:n, 0]
